# baseline (speedup 1.0000x reference)
import functools

import jax
import jax.numpy as jnp
from jax import lax
from jax.experimental import pallas as pl
from jax.experimental.pallas import tpu as pltpu
from jax.experimental.pallas import tpu_sc as plsc

_DIM = 1024
_ROWS = 8192
_NUM_CORES = 2
_NUM_SUBCORES = 16
_NW = _NUM_CORES * _NUM_SUBCORES
_ROWS_PER_W = _ROWS // _NW
_CHUNK = 40
_NBUF = 3
_CHUNKS = []
_off = 0
while _off < _ROWS_PER_W:
    _CHUNKS.append((_off, min(_CHUNK, _ROWS_PER_W - _off)))
    _off += _CHUNK
_NCHUNK = len(_CHUNKS)


@functools.partial(
    pl.kernel,
    out_type=jax.ShapeDtypeStruct((_ROWS, _DIM), jnp.float32),
    mesh=plsc.VectorSubcoreMesh(core_axis_name="c", subcore_axis_name="s"),
    scratch_types=[
        pltpu.VMEM((_NBUF, _CHUNK, _DIM), jnp.float32),
        pltpu.SemaphoreType.DMA,
        pltpu.SemaphoreType.DMA,
    ],
)
def _pos_embed_copy(emb_hbm, out_hbm, buf, rsem, wsem):
    wid = lax.axis_index("s") * _NUM_CORES + lax.axis_index("c")
    base = wid * _ROWS_PER_W

    def read(i):
        off, sz = _CHUNKS[i]
        return pltpu.async_copy(
            emb_hbm.at[pl.ds(base + off, sz)],
            buf.at[i % _NBUF, pl.ds(0, sz)],
            rsem,
        )

    def write(i):
        off, sz = _CHUNKS[i]
        return pltpu.async_copy(
            buf.at[i % _NBUF, pl.ds(0, sz)],
            out_hbm.at[pl.ds(base + off, sz)],
            wsem,
        )

    reads = {}
    writes = {}
    for i in range(min(_NBUF, _NCHUNK)):
        reads[i] = read(i)
    for i in range(_NCHUNK):
        reads[i].wait()
        writes[i] = write(i)
        nxt = i + _NBUF
        if nxt < _NCHUNK:
            writes[i].wait()
            reads[nxt] = read(nxt)
    for i in range(max(0, _NCHUNK - _NBUF), _NCHUNK):
        writes[i].wait()


def kernel(x, emb):
    del x
    return _pos_embed_copy(emb)[None, :, :]


# speedup vs baseline: 1.0013x; 1.0013x over previous
import functools

import jax
import jax.numpy as jnp
from jax import lax
from jax.experimental import pallas as pl
from jax.experimental.pallas import tpu as pltpu
from jax.experimental.pallas import tpu_sc as plsc

_DIM = 1024
_ROWS = 8192
_NUM_CORES = 2
_NUM_SUBCORES = 16
_NW = _NUM_CORES * _NUM_SUBCORES
_ROWS_PER_W = _ROWS // _NW
_NBUF = 2
_SIZES = [8, 24, 56, 56, 56, 40, 16]
assert sum(_SIZES) == _ROWS_PER_W
_BUF_ROWS = max(_SIZES)
_CHUNKS = []
_off = 0
for _sz in _SIZES:
    _CHUNKS.append((_off, _sz))
    _off += _sz
_NCHUNK = len(_CHUNKS)


@functools.partial(
    pl.kernel,
    out_type=jax.ShapeDtypeStruct((_ROWS, _DIM), jnp.float32),
    mesh=plsc.VectorSubcoreMesh(core_axis_name="c", subcore_axis_name="s"),
    scratch_types=[
        pltpu.VMEM((_NBUF, _BUF_ROWS, _DIM), jnp.float32),
        pltpu.SemaphoreType.DMA,
        pltpu.SemaphoreType.DMA,
    ],
)
def _pos_embed_copy(emb_hbm, out_hbm, buf, rsem, wsem):
    wid = lax.axis_index("s") * _NUM_CORES + lax.axis_index("c")
    base = wid * _ROWS_PER_W

    def read(i):
        off, sz = _CHUNKS[i]
        return pltpu.async_copy(
            emb_hbm.at[pl.ds(base + off, sz)],
            buf.at[i % _NBUF, pl.ds(0, sz)],
            rsem,
        )

    def write(i):
        off, sz = _CHUNKS[i]
        return pltpu.async_copy(
            buf.at[i % _NBUF, pl.ds(0, sz)],
            out_hbm.at[pl.ds(base + off, sz)],
            wsem,
        )

    reads = {}
    writes = {}
    for i in range(min(_NBUF, _NCHUNK)):
        reads[i] = read(i)
    for i in range(_NCHUNK):
        reads[i].wait()
        writes[i] = write(i)
        nxt = i + _NBUF
        if nxt < _NCHUNK:
            writes[i].wait()
            reads[nxt] = read(nxt)
    for i in range(max(0, _NCHUNK - _NBUF), _NCHUNK):
        writes[i].wait()


def kernel(x, emb):
    del x
    return _pos_embed_copy(emb)[None, :, :]


# speedup vs baseline: 1.0136x; 1.0123x over previous
import functools

import jax
import jax.numpy as jnp
from jax import lax
from jax.experimental import pallas as pl
from jax.experimental.pallas import tpu as pltpu
from jax.experimental.pallas import tpu_sc as plsc

_DIM = 1024
_ROWS = 8192
_NUM_CORES = 2
_NUM_SUBCORES = 16
_NW = _NUM_CORES * _NUM_SUBCORES
_ROWS_PER_W = _ROWS // _NW
_NBUF = 2
_SIZES = [56, 56, 56, 56, 32]
assert sum(_SIZES) == _ROWS_PER_W
_BUF_ROWS = max(_SIZES)
_CHUNKS = []
_off = 0
for _sz in _SIZES:
    _CHUNKS.append((_off, _sz))
    _off += _sz
_NCHUNK = len(_CHUNKS)


@functools.partial(
    pl.kernel,
    out_type=jax.ShapeDtypeStruct((_ROWS, _DIM), jnp.float32),
    mesh=plsc.VectorSubcoreMesh(core_axis_name="c", subcore_axis_name="s"),
    scratch_types=[
        pltpu.VMEM((_NBUF, _BUF_ROWS, _DIM), jnp.float32),
        pltpu.SemaphoreType.DMA,
        pltpu.SemaphoreType.DMA,
    ],
)
def _pos_embed_copy(emb_hbm, out_hbm, buf, rsem, wsem):
    wid = lax.axis_index("s") * _NUM_CORES + lax.axis_index("c")
    base = wid * _ROWS_PER_W

    def read(i):
        off, sz = _CHUNKS[i]
        return pltpu.async_copy(
            emb_hbm.at[pl.ds(base + off, sz)],
            buf.at[i % _NBUF, pl.ds(0, sz)],
            rsem,
        )

    def write(i):
        off, sz = _CHUNKS[i]
        return pltpu.async_copy(
            buf.at[i % _NBUF, pl.ds(0, sz)],
            out_hbm.at[pl.ds(base + off, sz)],
            wsem,
        )

    reads = {}
    writes = {}
    for i in range(min(_NBUF, _NCHUNK)):
        reads[i] = read(i)
    for i in range(_NCHUNK):
        reads[i].wait()
        writes[i] = write(i)
        nxt = i + _NBUF
        if nxt < _NCHUNK:
            writes[i].wait()
            reads[nxt] = read(nxt)
    for i in range(max(0, _NCHUNK - _NBUF), _NCHUNK):
        writes[i].wait()


def kernel(x, emb):
    del x
    return _pos_embed_copy(emb)[None, :, :]
